# baseline (speedup 1.0000x reference)
import jax
import jax.numpy as jnp
from jax.experimental import pallas as pl
from jax.experimental.pallas import tpu as pltpu

_KERNEL_MUL = 2.0
_KERNEL_NUM = 5
_NCHUNK = 4


def _mkmmd_kernel(src_hbm, tgt_hbm, out_ref, buf, csc, sem):
    nb, d = src_hbm.shape
    n = 2 * nb
    ch = nb // _NCHUNK

    def copies(k):
        return (
            pltpu.make_async_copy(src_hbm.at[pl.ds(k * ch, ch)],
                                  buf.at[pl.ds(k * ch, ch)], sem.at[k]),
            pltpu.make_async_copy(tgt_hbm.at[pl.ds(k * ch, ch)],
                                  buf.at[pl.ds(nb + k * ch, ch)], sem.at[k]),
        )

    for k in range(_NCHUNK):
        a, b = copies(k)
        a.start()
        b.start()

    row = jax.lax.broadcasted_iota(jnp.int32, (d, d), 0)
    col = jax.lax.broadcasted_iota(jnp.int32, (d, d), 1)
    tri = jnp.where(row <= col, 1.0, 0.0).astype(jnp.bfloat16)
    w = (d - jax.lax.broadcasted_iota(jnp.int32, (1, d), 1)).astype(jnp.float32)

    def chunk_sums(k):
        s = buf[k * ch:(k + 1) * ch]
        t = buf[nb + k * ch: nb + (k + 1) * ch]
        return (jnp.sum(s, axis=0, keepdims=True)
                + jnp.sum(t, axis=0, keepdims=True),
                jnp.sum(s * s, axis=0, keepdims=True)
                + jnp.sum(t * t, axis=0, keepdims=True))

    def chunk_mm(k):
        lo_r, hi_r = k * ch, (k + 1) * ch
        s_blk = buf[lo_r:hi_r]
        t_blk = buf[nb + lo_r: nb + hi_r]
        if k < _NCHUNK - 1:
            rs_blk = buf[lo_r + 1: hi_r + 1]
            rt_blk = buf[nb + lo_r + 1: nb + hi_r + 1]
        else:
            rs_blk = jnp.concatenate([buf[lo_r + 1: nb], buf[0:1]], axis=0)
            rt_blk = jnp.concatenate([buf[nb + lo_r + 1:], buf[nb:nb + 1]],
                                     axis=0)
        sq = jnp.concatenate(
            [s_blk - rs_blk, t_blk - rt_blk,
             s_blk - rt_blk, rs_blk - t_blk], axis=0)
        sq = sq * sq
        hi = sq.astype(jnp.bfloat16)
        lo = (sq - hi.astype(jnp.float32)).astype(jnp.bfloat16)
        c = (jnp.dot(hi, tri, preferred_element_type=jnp.float32)
             + jnp.dot(lo, tri, preferred_element_type=jnp.float32))
        csc[pl.ds(k * 4 * ch, 4 * ch), :] = c

    s1_parts, s2_parts = [], []
    for k in range(_NCHUNK):
        a, b = copies(k)
        a.wait()
        b.wait()
        p1, p2 = chunk_sums(k)
        s1_parts.append(p1)
        s2_parts.append(p2)
        if k >= 1:
            chunk_mm(k - 1)
    chunk_mm(_NCHUNK - 1)

    s1 = s1_parts[0]
    s2 = s2_parts[0]
    for k in range(1, _NCHUNK):
        s1 = s1 + s1_parts[k]
        s2 = s2 + s2_parts[k]
    colsum = (2.0 * n) * s2 - 2.0 * s1 * s1
    bw_sum = jnp.sum(w * colsum)
    bw = bw_sum / (n * n - n) / (_KERNEL_MUL ** (_KERNEL_NUM // 2))
    neg_inv = -1.0 / (bw * (_KERNEL_MUL ** (_KERNEL_NUM - 1)))

    half = 2 * ch
    acc = None
    for k in range(_NCHUNK):
        c = csc[pl.ds(k * 4 * ch, 4 * ch), :]
        z = jnp.exp(c * neg_inv)
        z2 = z * z
        z4 = z2 * z2
        z8 = z4 * z4
        z16 = z8 * z8
        ksum = z + z2 + z4 + z8 + z16
        comb = ksum[:half] - ksum[half:]
        acc = comb if acc is None else acc + comb
    total = jnp.sum(acc, axis=(0, 1), keepdims=True)
    out_ref[:, :] = total * (1.0 / (nb * d))


@jax.jit
def kernel(source, target):
    b, d = source.shape
    out = pl.pallas_call(
        _mkmmd_kernel,
        out_shape=jax.ShapeDtypeStruct((1, 1), jnp.float32),
        in_specs=[
            pl.BlockSpec(memory_space=pltpu.HBM),
            pl.BlockSpec(memory_space=pltpu.HBM),
        ],
        out_specs=pl.BlockSpec(memory_space=pltpu.VMEM),
        scratch_shapes=[
            pltpu.VMEM((2 * b, d), jnp.float32),
            pltpu.VMEM((4 * b, d), jnp.float32),
            pltpu.SemaphoreType.DMA((_NCHUNK,)),
        ],
    )(source, target)
    return out[0, 0]


# speedup vs baseline: 1.1656x; 1.1656x over previous
import jax
import jax.numpy as jnp
from jax.experimental import pallas as pl
from jax.experimental.pallas import tpu as pltpu

_KERNEL_MUL = 2.0
_KERNEL_NUM = 5


def _mkmmd_kernel(src_ref, tgt_ref, out_ref):
    src = src_ref[:]
    tgt = tgt_ref[:]
    nb, d = src.shape
    n = 2 * nb

    s1 = jnp.sum(src, axis=0, keepdims=True) + jnp.sum(tgt, axis=0, keepdims=True)
    s2 = (jnp.sum(src * src, axis=0, keepdims=True)
          + jnp.sum(tgt * tgt, axis=0, keepdims=True))
    colsum = (2.0 * n) * s2 - 2.0 * s1 * s1
    w = (d - jax.lax.broadcasted_iota(jnp.int32, (1, d), 1)).astype(jnp.float32)
    bw_sum = jnp.sum(w * colsum)
    bw = bw_sum / (n * n - n) / (_KERNEL_MUL ** (_KERNEL_NUM // 2))
    neg_inv = -1.0 / (bw * (_KERNEL_MUL ** (_KERNEL_NUM - 1)))

    rs = jnp.concatenate([src[1:], src[:1]], axis=0)
    rt = jnp.concatenate([tgt[1:], tgt[:1]], axis=0)

    row = jax.lax.broadcasted_iota(jnp.int32, (d, d), 0)
    col = jax.lax.broadcasted_iota(jnp.int32, (d, d), 1)
    tri = jnp.where(row <= col, 1.0, 0.0).astype(jnp.bfloat16)

    def kset(diff):
        sq = diff * diff
        hi = sq.astype(jnp.bfloat16)
        lo = (sq - hi.astype(jnp.float32)).astype(jnp.bfloat16)
        c = (jnp.dot(hi, tri, preferred_element_type=jnp.float32)
             + jnp.dot(lo, tri, preferred_element_type=jnp.float32))
        z = jnp.exp(c * neg_inv)
        z2 = z * z
        z4 = z2 * z2
        z8 = z4 * z4
        z16 = z8 * z8
        return z + z2 + z4 + z8 + z16

    comb = (kset(src - rs) + kset(tgt - rt)
            - kset(src - rt) - kset(rs - tgt))

    total = jnp.sum(comb, axis=(0, 1), keepdims=True)
    out_ref[:, :] = total * (1.0 / (nb * d))


@jax.jit
def kernel(source, target):
    out = pl.pallas_call(
        _mkmmd_kernel,
        out_shape=jax.ShapeDtypeStruct((1, 1), jnp.float32),
        in_specs=[
            pl.BlockSpec(memory_space=pltpu.VMEM),
            pl.BlockSpec(memory_space=pltpu.VMEM),
        ],
        out_specs=pl.BlockSpec(memory_space=pltpu.VMEM),
    )(source, target)
    return out[0, 0]


# speedup vs baseline: 1.1812x; 1.0134x over previous
import jax
import jax.numpy as jnp
from jax.experimental import pallas as pl
from jax.experimental.pallas import tpu as pltpu

_KERNEL_MUL = 2.0
_KERNEL_NUM = 5


def _mkmmd_kernel(src_ref, tgt_ref, out_ref):
    src = src_ref[:]
    tgt = tgt_ref[:]
    nb, d = src.shape
    n = 2 * nb

    s1 = jnp.sum(src, axis=0, keepdims=True) + jnp.sum(tgt, axis=0, keepdims=True)
    s2 = (jnp.sum(src * src, axis=0, keepdims=True)
          + jnp.sum(tgt * tgt, axis=0, keepdims=True))
    colsum = (2.0 * n) * s2 - 2.0 * s1 * s1
    w = (d - jax.lax.broadcasted_iota(jnp.int32, (1, d), 1)).astype(jnp.float32)
    bw_sum = jnp.sum(w * colsum)
    bw = bw_sum / (n * n - n) / (_KERNEL_MUL ** (_KERNEL_NUM // 2))
    neg_inv = -1.4426950408889634 / (bw * (_KERNEL_MUL ** (_KERNEL_NUM - 1)))

    rs = jnp.concatenate([src[1:], src[:1]], axis=0)
    rt = jnp.concatenate([tgt[1:], tgt[:1]], axis=0)

    row = jax.lax.broadcasted_iota(jnp.int32, (d, d), 0)
    col = jax.lax.broadcasted_iota(jnp.int32, (d, d), 1)
    tri = jnp.where(row <= col, 1.0, 0.0).astype(jnp.bfloat16)

    def kset(diff):
        sq = diff * diff
        hi = sq.astype(jnp.bfloat16)
        lo = (sq - hi.astype(jnp.float32)).astype(jnp.bfloat16)
        c = (jnp.dot(hi, tri, preferred_element_type=jnp.float32)
             + jnp.dot(lo, tri, preferred_element_type=jnp.float32))
        z = jnp.exp2(c * neg_inv)
        z2 = z * z
        z4 = z2 * z2
        z8 = z4 * z4
        z16 = z8 * z8
        return z + z2 + z4 + z8 + z16

    comb = (kset(src - rs) + kset(tgt - rt)
            - kset(src - rt) - kset(rs - tgt))

    total = jnp.sum(comb, axis=(0, 1), keepdims=True)
    out_ref[:, :] = total * (1.0 / (nb * d))


@jax.jit
def kernel(source, target):
    out = pl.pallas_call(
        _mkmmd_kernel,
        out_shape=jax.ShapeDtypeStruct((1, 1), jnp.float32),
        in_specs=[
            pl.BlockSpec(memory_space=pltpu.VMEM),
            pl.BlockSpec(memory_space=pltpu.VMEM),
        ],
        out_specs=pl.BlockSpec(memory_space=pltpu.VMEM),
    )(source, target)
    return out[0, 0]
